# baseline (speedup 1.0000x reference)
import functools

import jax
import jax.numpy as jnp
from jax import lax
from jax.experimental import pallas as pl
from jax.experimental.pallas import tpu as pltpu
from jax.experimental.pallas import tpu_sc as plsc

_N = 10000
_E = 320000
_DIN = 128
_DH = 64
_CW = 16
_NT = 16
_ROWS = 632
_N_PAD = _NT * _ROWS
_CH = 128
_TCH = 160
_E_PAD = _NT * _TCH * _CH
_NCH = _E_PAD // _CH
_NBUF = 4

_BLK = 2000
_GRID = _N // _BLK


def _make_seg_kernel(with_counts):
    mesh = plsc.VectorSubcoreMesh(core_axis_name="c", subcore_axis_name="s")
    out_type = [jax.ShapeDtypeStruct((_N_PAD, _DH), jnp.float32)] * 6
    if with_counts:
        out_type += [jax.ShapeDtypeStruct((_N_PAD, _CW), jnp.float32)] * 6
    scratch_types = [
        pltpu.VMEM((_TCH, _CH), jnp.int32),
        pltpu.VMEM((_TCH, _CH), jnp.int32),
        pltpu.VMEM((_NBUF, _CH, _DH), jnp.float32),
        pltpu.VMEM_SHARED((_N_PAD, _DH), jnp.float32),
        pltpu.VMEM((_CH, _CW), jnp.float32),
        pltpu.VMEM((_CH, _CW), jnp.float32),
        pltpu.VMEM_SHARED((_N_PAD, _CW), jnp.float32),
    ] + [pltpu.SemaphoreType.DMA] * (3 * _NBUF)

    def body(*refs):
        ys = refs[0:6]
        srcs = refs[6:12]
        dsts = refs[12:18]
        pos = 18
        s_out = refs[pos:pos + 6]
        pos += 6
        c_out = refs[pos:pos + 6] if with_counts else (None,) * 6
        if with_counts:
            pos += 6
        (src_v, dst_v, rows_v, acc, ones_v, cb_v, acc_c) = refs[pos:pos + 7]
        sems = refs[pos + 7:pos + 7 + _NBUF]
        ssems = refs[pos + 7 + _NBUF:pos + 7 + 2 * _NBUF]
        csems = refs[pos + 7 + 2 * _NBUF:pos + 7 + 3 * _NBUF]

        cid = lax.axis_index("c")
        sid = lax.axis_index("s")
        row0 = sid * _ROWS
        ch0 = sid * _TCH
        chunks = []
        r = 0
        while r < _ROWS:
            chunks.append((r, min(_CH, _ROWS - r)))
            r += _CH

        if with_counts:
            o16 = jnp.ones((16,), jnp.float32)

            def _oinit(i, c):
                ones_v[i, pl.ds(0, 16)] = o16
                return c
            lax.fori_loop(0, _CH, _oinit, 0)

        z16 = jnp.zeros((16,), jnp.float32)

        def _fill_zero():
            def _zr(i, c):
                for k in range(_DH // 16):
                    rows_v[0, i, pl.ds(k * 16, 16)] = z16
                cb_v[i, pl.ds(0, 16)] = z16
                return c
            lax.fori_loop(0, _CH, _zr, 0)

        def run_rel(y_ref, s_ref, d_ref, so_ref, co_ref):
            _fill_zero()
            for (r0, w) in chunks:
                pltpu.sync_copy(rows_v.at[0, pl.ds(0, w)],
                                acc.at[pl.ds(row0 + r0, w)])
                if with_counts:
                    pltpu.sync_copy(cb_v.at[pl.ds(0, w)],
                                    acc_c.at[pl.ds(row0 + r0, w)])
            plsc.subcore_barrier()

            pltpu.sync_copy(s_ref.at[pl.ds(ch0, _TCH)], src_v)
            pltpu.sync_copy(d_ref.at[pl.ds(ch0, _TCH)], dst_v)

            for b in range(_NBUF):
                pltpu.async_copy(y_ref.at[src_v.at[b]], rows_v.at[b], sems[b])

            def _push(j, b):
                pltpu.make_async_copy(y_ref.at[src_v.at[j]], rows_v.at[b],
                                      sems[b]).wait()
                pltpu.async_copy(rows_v.at[b], acc.at[dst_v.at[j]], ssems[b],
                                 add=True)
                if with_counts:
                    pltpu.async_copy(ones_v, acc_c.at[dst_v.at[j]], csems[b],
                                     add=True)

            def _drain(j, b):
                pltpu.make_async_copy(rows_v.at[b], acc.at[dst_v.at[j]],
                                      ssems[b]).wait()
                if with_counts:
                    pltpu.make_async_copy(ones_v, acc_c.at[dst_v.at[j]],
                                          csems[b]).wait()

            def _grp(g, c):
                for b in range(_NBUF):
                    j = g * _NBUF + b
                    _push(j, b)
                for b in range(_NBUF):
                    j = g * _NBUF + b
                    _drain(j, b)
                    pltpu.async_copy(y_ref.at[src_v.at[j + _NBUF]],
                                     rows_v.at[b], sems[b])
                return c
            lax.fori_loop(0, _TCH // _NBUF - 1, _grp, 0)
            for b in range(_NBUF):
                _push(_TCH - _NBUF + b, b)
            for b in range(_NBUF):
                _drain(_TCH - _NBUF + b, b)
            plsc.subcore_barrier()

            for (r0, w) in chunks:
                pltpu.sync_copy(acc.at[pl.ds(row0 + r0, w)],
                                rows_v.at[0, pl.ds(0, w)])
                pltpu.sync_copy(rows_v.at[0, pl.ds(0, w)],
                                so_ref.at[pl.ds(row0 + r0, w)])
                if with_counts:
                    pltpu.sync_copy(acc_c.at[pl.ds(row0 + r0, w)],
                                    cb_v.at[pl.ds(0, w)])
                    pltpu.sync_copy(cb_v.at[pl.ds(0, w)],
                                    co_ref.at[pl.ds(row0 + r0, w)])

        @pl.when(cid == 0)
        def _():
            for r in (0, 1, 2):
                run_rel(ys[r], srcs[r], dsts[r], s_out[r], c_out[r])

        @pl.when(cid == 1)
        def _():
            for r in (3, 4, 5):
                run_rel(ys[r], srcs[r], dsts[r], s_out[r], c_out[r])

    return pl.kernel(
        body, out_type=out_type, mesh=mesh, scratch_types=scratch_types,
        compiler_params=pltpu.CompilerParams(use_tc_tiling_on_sc=False))


_seg_with_counts = _make_seg_kernel(True)
_seg_no_counts = _make_seg_kernel(False)


def _mm2_body(x_ref, w1_ref, w2_ref, o1_ref, o2_ref):
    x = x_ref[...]
    o1_ref[...] = jnp.dot(x, w1_ref[...], preferred_element_type=jnp.float32)
    o2_ref[...] = jnp.dot(x, w2_ref[...], preferred_element_type=jnp.float32)


def _mm2(x, w1, w2):
    din = x.shape[1]
    return pl.pallas_call(
        _mm2_body,
        grid=(_GRID,),
        in_specs=[
            pl.BlockSpec((_BLK, din), lambda i: (i, 0)),
            pl.BlockSpec((din, _DH), lambda i: (0, 0)),
            pl.BlockSpec((din, _DH), lambda i: (0, 0)),
        ],
        out_specs=[
            pl.BlockSpec((_BLK, _DH), lambda i: (i, 0)),
            pl.BlockSpec((_BLK, _DH), lambda i: (i, 0)),
        ],
        out_shape=[jax.ShapeDtypeStruct((_N, _DH), jnp.float32)] * 2,
    )(x, w1, w2)


def _combine1_body(sa_ref, ca_ref, sb_ref, cb_ref, x_ref, wr_ref, b_ref,
                   wla_ref, wlb_ref, h_ref, ya_ref, yb_ref):
    ca = jnp.maximum(ca_ref[...][:, 0:1], 1.0)
    cb = jnp.maximum(cb_ref[...][:, 0:1], 1.0)
    agg = 0.5 * (sa_ref[...] / ca + sb_ref[...] / cb)
    h = agg + jnp.dot(x_ref[...], wr_ref[...],
                      preferred_element_type=jnp.float32) + b_ref[...]
    h = jnp.maximum(h, 0.0)
    h_ref[...] = h
    ya_ref[...] = jnp.dot(h, wla_ref[...], preferred_element_type=jnp.float32)
    yb_ref[...] = jnp.dot(h, wlb_ref[...], preferred_element_type=jnp.float32)


def _combine1(sa, ca, sb, cb, x, wr, b, wla, wlb):
    return pl.pallas_call(
        _combine1_body,
        grid=(_GRID,),
        in_specs=[
            pl.BlockSpec((_BLK, _DH), lambda i: (i, 0)),
            pl.BlockSpec((_BLK, _CW), lambda i: (i, 0)),
            pl.BlockSpec((_BLK, _DH), lambda i: (i, 0)),
            pl.BlockSpec((_BLK, _CW), lambda i: (i, 0)),
            pl.BlockSpec((_BLK, _DIN), lambda i: (i, 0)),
            pl.BlockSpec((_DIN, _DH), lambda i: (0, 0)),
            pl.BlockSpec((1, _DH), lambda i: (0, 0)),
            pl.BlockSpec((_DH, _DH), lambda i: (0, 0)),
            pl.BlockSpec((_DH, _DH), lambda i: (0, 0)),
        ],
        out_specs=[
            pl.BlockSpec((_BLK, _DH), lambda i: (i, 0)),
            pl.BlockSpec((_BLK, _DH), lambda i: (i, 0)),
            pl.BlockSpec((_BLK, _DH), lambda i: (i, 0)),
        ],
        out_shape=[jax.ShapeDtypeStruct((_N, _DH), jnp.float32)] * 3,
    )(sa, ca, sb, cb, x, wr, b, wla, wlb)


def _combine2_body(sa_ref, ca_ref, sb_ref, cb_ref, h_ref, x_ref, wr_ref,
                   wres_ref, b_ref, o_ref):
    ca = jnp.maximum(ca_ref[...][:, 0:1], 1.0)
    cb = jnp.maximum(cb_ref[...][:, 0:1], 1.0)
    agg = 0.5 * (sa_ref[...] / ca + sb_ref[...] / cb)
    o = (agg
         + jnp.dot(h_ref[...], wr_ref[...], preferred_element_type=jnp.float32)
         + jnp.dot(x_ref[...], wres_ref[...], preferred_element_type=jnp.float32)
         + b_ref[...])
    n = jnp.sqrt(jnp.sum(o * o, axis=1, keepdims=True))
    o_ref[...] = o / jnp.maximum(n, 1e-12)


def _combine2(sa, ca, sb, cb, h, x, wr, wres, b):
    return pl.pallas_call(
        _combine2_body,
        grid=(_GRID,),
        in_specs=[
            pl.BlockSpec((_BLK, _DH), lambda i: (i, 0)),
            pl.BlockSpec((_BLK, _CW), lambda i: (i, 0)),
            pl.BlockSpec((_BLK, _DH), lambda i: (i, 0)),
            pl.BlockSpec((_BLK, _CW), lambda i: (i, 0)),
            pl.BlockSpec((_BLK, _DH), lambda i: (i, 0)),
            pl.BlockSpec((_BLK, _DIN), lambda i: (i, 0)),
            pl.BlockSpec((_DH, _DH), lambda i: (0, 0)),
            pl.BlockSpec((_DIN, _DH), lambda i: (0, 0)),
            pl.BlockSpec((1, _DH), lambda i: (0, 0)),
        ],
        out_specs=pl.BlockSpec((_BLK, _DH), lambda i: (i, 0)),
        out_shape=jax.ShapeDtypeStruct((_N, _DH), jnp.float32),
    )(sa, ca, sb, cb, h, x, wr, wres, b)


def _prep_idx(src, dst):
    src = src.astype(jnp.int32)
    dst = dst.astype(jnp.int32)
    pad = _E_PAD - _E
    src_p = jnp.concatenate([src, jnp.zeros((pad,), jnp.int32)])
    dst_p = jnp.concatenate([dst, jnp.full((pad,), _N, jnp.int32)])
    return src_p.reshape(_NCH, _CH), dst_p.reshape(_NCH, _CH)


_REL = ('mc', 'dc', 'cm', 'dm', 'md', 'cd')
_SRCTY = {'mc': 'm', 'cm': 'c', 'dm': 'd', 'md': 'm', 'dc': 'd', 'cd': 'c'}
_DST_OF = {'c': ('mc', 'dc'), 'm': ('cm', 'dm'), 'd': ('md', 'cd')}


def kernel(x_c, x_m, x_d, e_cm, e_md, e_cd, params):
    P1, P2, Pr = params['l1'], params['l2'], params['res']

    edge = {
        'mc': (e_cm[1], e_cm[0]), 'cm': (e_cm[0], e_cm[1]),
        'dm': (e_md[1], e_md[0]), 'md': (e_md[0], e_md[1]),
        'dc': (e_cd[1], e_cd[0]), 'cd': (e_cd[0], e_cd[1]),
    }
    idx = {r: _prep_idx(*edge[r]) for r in _REL}
    X = {'c': x_c, 'm': x_m, 'd': x_d}

    y = {}
    y['cm'], y['cd'] = _mm2(x_c, P1['cm']['Wl'], P1['cd']['Wl'])
    y['mc'], y['md'] = _mm2(x_m, P1['mc']['Wl'], P1['md']['Wl'])
    y['dm'], y['dc'] = _mm2(x_d, P1['dm']['Wl'], P1['dc']['Wl'])

    args = ([y[r] for r in _REL]
            + [idx[r][0] for r in _REL]
            + [idx[r][1] for r in _REL])
    outs = _seg_with_counts(*args)
    S1 = {r: outs[i] for i, r in enumerate(_REL)}
    C = {r: outs[6 + i] for i, r in enumerate(_REL)}

    H, y2 = {}, {}
    for t in ('c', 'm', 'd'):
        a, b = _DST_OF[t]
        wr = 0.5 * (P1[a]['Wr'] + P1[b]['Wr'])
        bb = (0.5 * (P1[a]['bl'] + P1[b]['bl'])).reshape(1, _DH)
        ra, rb = [r for r in _REL if _SRCTY[r] == t]
        H[t], y2[ra], y2[rb] = _combine1(
            S1[a], C[a], S1[b], C[b], X[t], wr, bb,
            P2[ra]['Wl'], P2[rb]['Wl'])

    args2 = ([y2[r] for r in _REL]
             + [idx[r][0] for r in _REL]
             + [idx[r][1] for r in _REL])
    outs2 = _seg_no_counts(*args2)
    S2 = {r: outs2[i] for i, r in enumerate(_REL)}

    O = {}
    for t in ('c', 'm', 'd'):
        a, b = _DST_OF[t]
        wr = 0.5 * (P2[a]['Wr'] + P2[b]['Wr'])
        bb = (0.5 * (P2[a]['bl'] + P2[b]['bl']) + Pr[t]['b']).reshape(1, _DH)
        O[t] = _combine2(S2[a], C[a], S2[b], C[b], H[t], X[t],
                         wr, Pr[t]['W'], bb)

    return O['c'], O['m'], O['d']


# speedup vs baseline: 1.4965x; 1.4965x over previous
import functools

import jax
import jax.numpy as jnp
from jax import lax
from jax.experimental import pallas as pl
from jax.experimental.pallas import tpu as pltpu
from jax.experimental.pallas import tpu_sc as plsc

_N = 10000
_E = 320000
_DIN = 128
_DH = 64
_CW = 16
_NT = 16
_ROWS = 632
_N_PAD = _NT * _ROWS
_CH = 128
_TCH = 160
_E_PAD = _NT * _TCH * _CH
_NCH = _E_PAD // _CH
_NBUF = 2
_ST = 40
_NSTAGE = _TCH // _ST

_BLK = 2000
_GRID = _N // _BLK


def _make_seg_kernel(with_counts):
    mesh = plsc.VectorSubcoreMesh(core_axis_name="c", subcore_axis_name="s")
    out_type = [jax.ShapeDtypeStruct((_N_PAD, _DH), jnp.float32)] * 6
    if with_counts:
        out_type += [jax.ShapeDtypeStruct((_N_PAD, _CW), jnp.float32)] * 6
    scratch_types = [
        pltpu.VMEM((_ST, _CH), jnp.int32),
        pltpu.VMEM((_ST, _CH), jnp.int32),
        pltpu.VMEM((_NBUF, _CH, _DH), jnp.float32),
        pltpu.VMEM_SHARED((_N_PAD, _DH), jnp.float32),
        pltpu.VMEM_SHARED((_N_PAD, _DH), jnp.float32),
        pltpu.VMEM((_CH, _CW), jnp.float32),
        pltpu.VMEM((_CH, _CW), jnp.float32),
        pltpu.VMEM_SHARED((_N_PAD, _CW), jnp.float32),
    ] + [pltpu.SemaphoreType.DMA] * _NBUF

    def body(*refs):
        ys = refs[0:6]
        srcs = refs[6:12]
        dsts = refs[12:18]
        pos = 18
        s_out = refs[pos:pos + 6]
        pos += 6
        c_out = refs[pos:pos + 6] if with_counts else (None,) * 6
        if with_counts:
            pos += 6
        (src_v, dst_v, rows_v, acc, tbl, ones_v, cb_v, acc_c) = refs[pos:pos + 8]
        sems = refs[pos + 8:]

        cid = lax.axis_index("c")
        sid = lax.axis_index("s")
        row0 = sid * _ROWS
        ch0 = sid * _TCH
        chunks = []
        r = 0
        while r < _ROWS:
            chunks.append((r, min(_CH, _ROWS - r)))
            r += _CH

        if with_counts:
            o16 = jnp.ones((16,), jnp.float32)

            def _oinit(i, c):
                ones_v[i, pl.ds(0, 16)] = o16
                return c
            lax.fori_loop(0, _CH, _oinit, 0)

        z16 = jnp.zeros((16,), jnp.float32)

        def _fill_zero():
            def _zr(i, c):
                for k in range(_DH // 16):
                    rows_v[0, i, pl.ds(k * 16, 16)] = z16
                cb_v[i, pl.ds(0, 16)] = z16
                return c
            lax.fori_loop(0, _CH, _zr, 0)

        def run_rel(y_ref, s_ref, d_ref, so_ref, co_ref):
            for (r0, w) in chunks:
                pltpu.sync_copy(y_ref.at[pl.ds(row0 + r0, w)],
                                rows_v.at[0, pl.ds(0, w)])
                pltpu.sync_copy(rows_v.at[0, pl.ds(0, w)],
                                tbl.at[pl.ds(row0 + r0, w)])
            _fill_zero()
            for (r0, w) in chunks:
                pltpu.sync_copy(rows_v.at[0, pl.ds(0, w)],
                                acc.at[pl.ds(row0 + r0, w)])
                if with_counts:
                    pltpu.sync_copy(cb_v.at[pl.ds(0, w)],
                                    acc_c.at[pl.ds(row0 + r0, w)])
            plsc.subcore_barrier()

            def _wait_scatter(j, b):
                pltpu.make_async_copy(tbl.at[src_v.at[j]], rows_v.at[b],
                                      sems[b]).wait()
                pltpu.sync_copy(rows_v.at[b], acc.at[dst_v.at[j]], add=True)
                if with_counts:
                    pltpu.sync_copy(ones_v, acc_c.at[dst_v.at[j]], add=True)

            def _stage(s, c):
                pltpu.sync_copy(s_ref.at[pl.ds(ch0 + s * _ST, _ST)], src_v)
                pltpu.sync_copy(d_ref.at[pl.ds(ch0 + s * _ST, _ST)], dst_v)
                for b in range(_NBUF):
                    pltpu.async_copy(tbl.at[src_v.at[b]], rows_v.at[b],
                                     sems[b])

                def _grp(g, c2):
                    for b in range(_NBUF):
                        j = g * _NBUF + b
                        _wait_scatter(j, b)
                        pltpu.async_copy(tbl.at[src_v.at[j + _NBUF]],
                                         rows_v.at[b], sems[b])
                    return c2
                lax.fori_loop(0, _ST // _NBUF - 1, _grp, 0)
                for b in range(_NBUF):
                    _wait_scatter(_ST - _NBUF + b, b)
                return c
            lax.fori_loop(0, _NSTAGE, _stage, 0)
            plsc.subcore_barrier()

            for (r0, w) in chunks:
                pltpu.sync_copy(acc.at[pl.ds(row0 + r0, w)],
                                rows_v.at[0, pl.ds(0, w)])
                pltpu.sync_copy(rows_v.at[0, pl.ds(0, w)],
                                so_ref.at[pl.ds(row0 + r0, w)])
                if with_counts:
                    pltpu.sync_copy(acc_c.at[pl.ds(row0 + r0, w)],
                                    cb_v.at[pl.ds(0, w)])
                    pltpu.sync_copy(cb_v.at[pl.ds(0, w)],
                                    co_ref.at[pl.ds(row0 + r0, w)])

        @pl.when(cid == 0)
        def _():
            for r in (0, 1, 2):
                run_rel(ys[r], srcs[r], dsts[r], s_out[r], c_out[r])

        @pl.when(cid == 1)
        def _():
            for r in (3, 4, 5):
                run_rel(ys[r], srcs[r], dsts[r], s_out[r], c_out[r])

    return pl.kernel(
        body, out_type=out_type, mesh=mesh, scratch_types=scratch_types,
        compiler_params=pltpu.CompilerParams(use_tc_tiling_on_sc=False))


_seg_with_counts = _make_seg_kernel(True)
_seg_no_counts = _make_seg_kernel(False)


def _mm2_body(x_ref, w1_ref, w2_ref, o1_ref, o2_ref):
    x = x_ref[...]
    o1_ref[...] = jnp.dot(x, w1_ref[...], preferred_element_type=jnp.float32)
    o2_ref[...] = jnp.dot(x, w2_ref[...], preferred_element_type=jnp.float32)


def _mm2(x, w1, w2):
    din = x.shape[1]
    return pl.pallas_call(
        _mm2_body,
        grid=(_GRID,),
        in_specs=[
            pl.BlockSpec((_BLK, din), lambda i: (i, 0)),
            pl.BlockSpec((din, _DH), lambda i: (0, 0)),
            pl.BlockSpec((din, _DH), lambda i: (0, 0)),
        ],
        out_specs=[
            pl.BlockSpec((_BLK, _DH), lambda i: (i, 0)),
            pl.BlockSpec((_BLK, _DH), lambda i: (i, 0)),
        ],
        out_shape=[jax.ShapeDtypeStruct((_N_PAD, _DH), jnp.float32)] * 2,
    )(x, w1, w2)


def _combine1_body(sa_ref, ca_ref, sb_ref, cb_ref, x_ref, wr_ref, b_ref,
                   wla_ref, wlb_ref, h_ref, ya_ref, yb_ref):
    ca = jnp.maximum(ca_ref[...][:, 0:1], 1.0)
    cb = jnp.maximum(cb_ref[...][:, 0:1], 1.0)
    agg = 0.5 * (sa_ref[...] / ca + sb_ref[...] / cb)
    h = agg + jnp.dot(x_ref[...], wr_ref[...],
                      preferred_element_type=jnp.float32) + b_ref[...]
    h = jnp.maximum(h, 0.0)
    h_ref[...] = h
    ya_ref[...] = jnp.dot(h, wla_ref[...], preferred_element_type=jnp.float32)
    yb_ref[...] = jnp.dot(h, wlb_ref[...], preferred_element_type=jnp.float32)


def _combine1(sa, ca, sb, cb, x, wr, b, wla, wlb):
    return pl.pallas_call(
        _combine1_body,
        grid=(_GRID,),
        in_specs=[
            pl.BlockSpec((_BLK, _DH), lambda i: (i, 0)),
            pl.BlockSpec((_BLK, _CW), lambda i: (i, 0)),
            pl.BlockSpec((_BLK, _DH), lambda i: (i, 0)),
            pl.BlockSpec((_BLK, _CW), lambda i: (i, 0)),
            pl.BlockSpec((_BLK, _DIN), lambda i: (i, 0)),
            pl.BlockSpec((_DIN, _DH), lambda i: (0, 0)),
            pl.BlockSpec((1, _DH), lambda i: (0, 0)),
            pl.BlockSpec((_DH, _DH), lambda i: (0, 0)),
            pl.BlockSpec((_DH, _DH), lambda i: (0, 0)),
        ],
        out_specs=[
            pl.BlockSpec((_BLK, _DH), lambda i: (i, 0)),
            pl.BlockSpec((_BLK, _DH), lambda i: (i, 0)),
            pl.BlockSpec((_BLK, _DH), lambda i: (i, 0)),
        ],
        out_shape=[jax.ShapeDtypeStruct((_N, _DH), jnp.float32),
                   jax.ShapeDtypeStruct((_N_PAD, _DH), jnp.float32),
                   jax.ShapeDtypeStruct((_N_PAD, _DH), jnp.float32)],
    )(sa, ca, sb, cb, x, wr, b, wla, wlb)


def _combine2_body(sa_ref, ca_ref, sb_ref, cb_ref, h_ref, x_ref, wr_ref,
                   wres_ref, b_ref, o_ref):
    ca = jnp.maximum(ca_ref[...][:, 0:1], 1.0)
    cb = jnp.maximum(cb_ref[...][:, 0:1], 1.0)
    agg = 0.5 * (sa_ref[...] / ca + sb_ref[...] / cb)
    o = (agg
         + jnp.dot(h_ref[...], wr_ref[...], preferred_element_type=jnp.float32)
         + jnp.dot(x_ref[...], wres_ref[...], preferred_element_type=jnp.float32)
         + b_ref[...])
    n = jnp.sqrt(jnp.sum(o * o, axis=1, keepdims=True))
    o_ref[...] = o / jnp.maximum(n, 1e-12)


def _combine2(sa, ca, sb, cb, h, x, wr, wres, b):
    return pl.pallas_call(
        _combine2_body,
        grid=(_GRID,),
        in_specs=[
            pl.BlockSpec((_BLK, _DH), lambda i: (i, 0)),
            pl.BlockSpec((_BLK, _CW), lambda i: (i, 0)),
            pl.BlockSpec((_BLK, _DH), lambda i: (i, 0)),
            pl.BlockSpec((_BLK, _CW), lambda i: (i, 0)),
            pl.BlockSpec((_BLK, _DH), lambda i: (i, 0)),
            pl.BlockSpec((_BLK, _DIN), lambda i: (i, 0)),
            pl.BlockSpec((_DH, _DH), lambda i: (0, 0)),
            pl.BlockSpec((_DIN, _DH), lambda i: (0, 0)),
            pl.BlockSpec((1, _DH), lambda i: (0, 0)),
        ],
        out_specs=pl.BlockSpec((_BLK, _DH), lambda i: (i, 0)),
        out_shape=jax.ShapeDtypeStruct((_N, _DH), jnp.float32),
    )(sa, ca, sb, cb, h, x, wr, wres, b)


def _prep_idx(src, dst):
    src = src.astype(jnp.int32)
    dst = dst.astype(jnp.int32)
    pad = _E_PAD - _E
    src_p = jnp.concatenate([src, jnp.zeros((pad,), jnp.int32)])
    dst_p = jnp.concatenate([dst, jnp.full((pad,), _N, jnp.int32)])
    return src_p.reshape(_NCH, _CH), dst_p.reshape(_NCH, _CH)


_REL = ('mc', 'dc', 'cm', 'dm', 'md', 'cd')
_SRCTY = {'mc': 'm', 'cm': 'c', 'dm': 'd', 'md': 'm', 'dc': 'd', 'cd': 'c'}
_DST_OF = {'c': ('mc', 'dc'), 'm': ('cm', 'dm'), 'd': ('md', 'cd')}


def kernel(x_c, x_m, x_d, e_cm, e_md, e_cd, params):
    P1, P2, Pr = params['l1'], params['l2'], params['res']

    edge = {
        'mc': (e_cm[1], e_cm[0]), 'cm': (e_cm[0], e_cm[1]),
        'dm': (e_md[1], e_md[0]), 'md': (e_md[0], e_md[1]),
        'dc': (e_cd[1], e_cd[0]), 'cd': (e_cd[0], e_cd[1]),
    }
    idx = {r: _prep_idx(*edge[r]) for r in _REL}
    X = {'c': x_c, 'm': x_m, 'd': x_d}

    y = {}
    y['cm'], y['cd'] = _mm2(x_c, P1['cm']['Wl'], P1['cd']['Wl'])
    y['mc'], y['md'] = _mm2(x_m, P1['mc']['Wl'], P1['md']['Wl'])
    y['dm'], y['dc'] = _mm2(x_d, P1['dm']['Wl'], P1['dc']['Wl'])

    args = ([y[r] for r in _REL]
            + [idx[r][0] for r in _REL]
            + [idx[r][1] for r in _REL])
    outs = _seg_with_counts(*args)
    S1 = {r: outs[i] for i, r in enumerate(_REL)}
    C = {r: outs[6 + i] for i, r in enumerate(_REL)}

    H, y2 = {}, {}
    for t in ('c', 'm', 'd'):
        a, b = _DST_OF[t]
        wr = 0.5 * (P1[a]['Wr'] + P1[b]['Wr'])
        bb = (0.5 * (P1[a]['bl'] + P1[b]['bl'])).reshape(1, _DH)
        ra, rb = [r for r in _REL if _SRCTY[r] == t]
        H[t], y2[ra], y2[rb] = _combine1(
            S1[a], C[a], S1[b], C[b], X[t], wr, bb,
            P2[ra]['Wl'], P2[rb]['Wl'])

    args2 = ([y2[r] for r in _REL]
             + [idx[r][0] for r in _REL]
             + [idx[r][1] for r in _REL])
    outs2 = _seg_no_counts(*args2)
    S2 = {r: outs2[i] for i, r in enumerate(_REL)}

    O = {}
    for t in ('c', 'm', 'd'):
        a, b = _DST_OF[t]
        wr = 0.5 * (P2[a]['Wr'] + P2[b]['Wr'])
        bb = (0.5 * (P2[a]['bl'] + P2[b]['bl']) + Pr[t]['b']).reshape(1, _DH)
        O[t] = _combine2(S2[a], C[a], S2[b], C[b], H[t], X[t],
                         wr, Pr[t]['W'], bb)

    return O['c'], O['m'], O['d']


# speedup vs baseline: 1.5280x; 1.0210x over previous
import jax
import jax.numpy as jnp
from jax import lax
from jax.experimental import pallas as pl
from jax.experimental.pallas import tpu as pltpu
from jax.experimental.pallas import tpu_sc as plsc

_N = 10000
_E = 320000
_DIN = 128
_DH = 64
_W1 = 80
_NT = 16
_ROWS = 632
_N_PAD = _NT * _ROWS
_CH = 128
_TCH = 160
_E_PAD = _NT * _TCH * _CH
_NCH = _E_PAD // _CH
_NBUF = 2
_ST = 32
_NSTAGE = _TCH // _ST

_BLK = 2000
_GRID = _N // _BLK


def _make_seg_kernel(width):
    mesh = plsc.VectorSubcoreMesh(core_axis_name="c", subcore_axis_name="s")
    out_type = [jax.ShapeDtypeStruct((_N_PAD, width), jnp.float32)] * 6
    scratch_types = [
        pltpu.VMEM((_ST, _CH), jnp.int32),
        pltpu.VMEM((_ST, _CH), jnp.int32),
        pltpu.VMEM((_NBUF, _CH, width), jnp.float32),
        pltpu.VMEM_SHARED((_N_PAD, width), jnp.float32),
        pltpu.VMEM_SHARED((_N_PAD, width), jnp.float32),
    ] + [pltpu.SemaphoreType.DMA] * _NBUF

    def body(*refs):
        ys = refs[0:6]
        srcs = refs[6:12]
        dsts = refs[12:18]
        s_out = refs[18:24]
        (src_v, dst_v, rows_v, acc, tbl) = refs[24:29]
        sems = refs[29:]

        cid = lax.axis_index("c")
        sid = lax.axis_index("s")
        row0 = sid * _ROWS
        ch0 = sid * _TCH
        chunks = []
        r = 0
        while r < _ROWS:
            chunks.append((r, min(_CH, _ROWS - r)))
            r += _CH

        z16 = jnp.zeros((16,), jnp.float32)

        def _fill_zero():
            def _zr(i, c):
                for k in range(width // 16):
                    rows_v[0, i, pl.ds(k * 16, 16)] = z16
                return c
            lax.fori_loop(0, _CH, _zr, 0)

        def run_rel(y_ref, s_ref, d_ref, so_ref):
            for (r0, w) in chunks:
                pltpu.sync_copy(y_ref.at[pl.ds(row0 + r0, w)],
                                rows_v.at[0, pl.ds(0, w)])
                pltpu.sync_copy(rows_v.at[0, pl.ds(0, w)],
                                tbl.at[pl.ds(row0 + r0, w)])
            _fill_zero()
            for (r0, w) in chunks:
                pltpu.sync_copy(rows_v.at[0, pl.ds(0, w)],
                                acc.at[pl.ds(row0 + r0, w)])
            plsc.subcore_barrier()

            def _wait_scatter(j, b):
                pltpu.make_async_copy(tbl.at[src_v.at[j]], rows_v.at[b],
                                      sems[b]).wait()
                pltpu.sync_copy(rows_v.at[b], acc.at[dst_v.at[j]], add=True)

            def _stage(s, c):
                pltpu.sync_copy(s_ref.at[pl.ds(ch0 + s * _ST, _ST)], src_v)
                pltpu.sync_copy(d_ref.at[pl.ds(ch0 + s * _ST, _ST)], dst_v)
                for b in range(_NBUF):
                    pltpu.async_copy(tbl.at[src_v.at[b]], rows_v.at[b],
                                     sems[b])

                def _grp(g, c2):
                    for b in range(_NBUF):
                        j = g * _NBUF + b
                        _wait_scatter(j, b)
                        pltpu.async_copy(tbl.at[src_v.at[j + _NBUF]],
                                         rows_v.at[b], sems[b])
                    return c2
                lax.fori_loop(0, _ST // _NBUF - 1, _grp, 0)
                for b in range(_NBUF):
                    _wait_scatter(_ST - _NBUF + b, b)
                return c
            lax.fori_loop(0, _NSTAGE, _stage, 0)
            plsc.subcore_barrier()

            for (r0, w) in chunks:
                pltpu.sync_copy(acc.at[pl.ds(row0 + r0, w)],
                                rows_v.at[0, pl.ds(0, w)])
                pltpu.sync_copy(rows_v.at[0, pl.ds(0, w)],
                                so_ref.at[pl.ds(row0 + r0, w)])

        @pl.when(cid == 0)
        def _():
            for r in (0, 1, 2):
                run_rel(ys[r], srcs[r], dsts[r], s_out[r])

        @pl.when(cid == 1)
        def _():
            for r in (3, 4, 5):
                run_rel(ys[r], srcs[r], dsts[r], s_out[r])

    return pl.kernel(
        body, out_type=out_type, mesh=mesh, scratch_types=scratch_types,
        compiler_params=pltpu.CompilerParams(use_tc_tiling_on_sc=False))


_seg_l1 = _make_seg_kernel(_W1)
_seg_l2 = _make_seg_kernel(_DH)


def _mm2_body(x_ref, w1_ref, w2_ref, o1_ref, o2_ref):
    x = x_ref[...]
    o1_ref[:, 0:_DH] = jnp.dot(x, w1_ref[...],
                               preferred_element_type=jnp.float32)
    o1_ref[:, _DH:_W1] = jnp.ones((_BLK, _W1 - _DH), jnp.float32)
    o2_ref[:, 0:_DH] = jnp.dot(x, w2_ref[...],
                               preferred_element_type=jnp.float32)
    o2_ref[:, _DH:_W1] = jnp.ones((_BLK, _W1 - _DH), jnp.float32)


def _mm2(x, w1, w2):
    return pl.pallas_call(
        _mm2_body,
        grid=(_GRID,),
        in_specs=[
            pl.BlockSpec((_BLK, _DIN), lambda i: (i, 0)),
            pl.BlockSpec((_DIN, _DH), lambda i: (0, 0)),
            pl.BlockSpec((_DIN, _DH), lambda i: (0, 0)),
        ],
        out_specs=[
            pl.BlockSpec((_BLK, _W1), lambda i: (i, 0)),
            pl.BlockSpec((_BLK, _W1), lambda i: (i, 0)),
        ],
        out_shape=[jax.ShapeDtypeStruct((_N_PAD, _W1), jnp.float32)] * 2,
    )(x, w1, w2)


def _combine1_body(sa_ref, sb_ref, x_ref, wr_ref, b_ref,
                   wla_ref, wlb_ref, h_ref, ya_ref, yb_ref):
    sa = sa_ref[...]
    sb = sb_ref[...]
    ca = jnp.maximum(sa[:, _DH:_DH + 1], 1.0)
    cb = jnp.maximum(sb[:, _DH:_DH + 1], 1.0)
    agg = 0.5 * (sa[:, 0:_DH] / ca + sb[:, 0:_DH] / cb)
    h = agg + jnp.dot(x_ref[...], wr_ref[...],
                      preferred_element_type=jnp.float32) + b_ref[...]
    h = jnp.maximum(h, 0.0)
    h_ref[...] = h
    ya_ref[...] = jnp.dot(h, wla_ref[...], preferred_element_type=jnp.float32)
    yb_ref[...] = jnp.dot(h, wlb_ref[...], preferred_element_type=jnp.float32)


def _combine1(sa, sb, x, wr, b, wla, wlb):
    return pl.pallas_call(
        _combine1_body,
        grid=(_GRID,),
        in_specs=[
            pl.BlockSpec((_BLK, _W1), lambda i: (i, 0)),
            pl.BlockSpec((_BLK, _W1), lambda i: (i, 0)),
            pl.BlockSpec((_BLK, _DIN), lambda i: (i, 0)),
            pl.BlockSpec((_DIN, _DH), lambda i: (0, 0)),
            pl.BlockSpec((1, _DH), lambda i: (0, 0)),
            pl.BlockSpec((_DH, _DH), lambda i: (0, 0)),
            pl.BlockSpec((_DH, _DH), lambda i: (0, 0)),
        ],
        out_specs=[
            pl.BlockSpec((_BLK, _DH), lambda i: (i, 0)),
            pl.BlockSpec((_BLK, _DH), lambda i: (i, 0)),
            pl.BlockSpec((_BLK, _DH), lambda i: (i, 0)),
        ],
        out_shape=[jax.ShapeDtypeStruct((_N, _DH), jnp.float32),
                   jax.ShapeDtypeStruct((_N_PAD, _DH), jnp.float32),
                   jax.ShapeDtypeStruct((_N_PAD, _DH), jnp.float32)],
    )(sa, sb, x, wr, b, wla, wlb)


def _combine2_body(sa_ref, s1a_ref, sb_ref, s1b_ref, h_ref, x_ref, wr_ref,
                   wres_ref, b_ref, o_ref):
    ca = jnp.maximum(s1a_ref[...][:, _DH:_DH + 1], 1.0)
    cb = jnp.maximum(s1b_ref[...][:, _DH:_DH + 1], 1.0)
    agg = 0.5 * (sa_ref[...] / ca + sb_ref[...] / cb)
    o = (agg
         + jnp.dot(h_ref[...], wr_ref[...], preferred_element_type=jnp.float32)
         + jnp.dot(x_ref[...], wres_ref[...], preferred_element_type=jnp.float32)
         + b_ref[...])
    n = jnp.sqrt(jnp.sum(o * o, axis=1, keepdims=True))
    o_ref[...] = o / jnp.maximum(n, 1e-12)


def _combine2(sa, s1a, sb, s1b, h, x, wr, wres, b):
    return pl.pallas_call(
        _combine2_body,
        grid=(_GRID,),
        in_specs=[
            pl.BlockSpec((_BLK, _DH), lambda i: (i, 0)),
            pl.BlockSpec((_BLK, _W1), lambda i: (i, 0)),
            pl.BlockSpec((_BLK, _DH), lambda i: (i, 0)),
            pl.BlockSpec((_BLK, _W1), lambda i: (i, 0)),
            pl.BlockSpec((_BLK, _DH), lambda i: (i, 0)),
            pl.BlockSpec((_BLK, _DIN), lambda i: (i, 0)),
            pl.BlockSpec((_DH, _DH), lambda i: (0, 0)),
            pl.BlockSpec((_DIN, _DH), lambda i: (0, 0)),
            pl.BlockSpec((1, _DH), lambda i: (0, 0)),
        ],
        out_specs=pl.BlockSpec((_BLK, _DH), lambda i: (i, 0)),
        out_shape=jax.ShapeDtypeStruct((_N, _DH), jnp.float32),
    )(sa, s1a, sb, s1b, h, x, wr, wres, b)


def _prep_idx(src, dst):
    src = src.astype(jnp.int32)
    dst = dst.astype(jnp.int32)
    pad = _E_PAD - _E
    src_p = jnp.concatenate([src, jnp.zeros((pad,), jnp.int32)])
    dst_p = jnp.concatenate([dst, jnp.full((pad,), _N, jnp.int32)])
    return src_p.reshape(_NCH, _CH), dst_p.reshape(_NCH, _CH)


_REL = ('mc', 'dc', 'cm', 'dm', 'md', 'cd')
_SRCTY = {'mc': 'm', 'cm': 'c', 'dm': 'd', 'md': 'm', 'dc': 'd', 'cd': 'c'}
_DST_OF = {'c': ('mc', 'dc'), 'm': ('cm', 'dm'), 'd': ('md', 'cd')}


def kernel(x_c, x_m, x_d, e_cm, e_md, e_cd, params):
    P1, P2, Pr = params['l1'], params['l2'], params['res']

    edge = {
        'mc': (e_cm[1], e_cm[0]), 'cm': (e_cm[0], e_cm[1]),
        'dm': (e_md[1], e_md[0]), 'md': (e_md[0], e_md[1]),
        'dc': (e_cd[1], e_cd[0]), 'cd': (e_cd[0], e_cd[1]),
    }
    idx = {r: _prep_idx(*edge[r]) for r in _REL}
    X = {'c': x_c, 'm': x_m, 'd': x_d}

    y = {}
    y['cm'], y['cd'] = _mm2(x_c, P1['cm']['Wl'], P1['cd']['Wl'])
    y['mc'], y['md'] = _mm2(x_m, P1['mc']['Wl'], P1['md']['Wl'])
    y['dm'], y['dc'] = _mm2(x_d, P1['dm']['Wl'], P1['dc']['Wl'])

    args = ([y[r] for r in _REL]
            + [idx[r][0] for r in _REL]
            + [idx[r][1] for r in _REL])
    outs = _seg_l1(*args)
    S1 = {r: outs[i] for i, r in enumerate(_REL)}

    H, y2 = {}, {}
    for t in ('c', 'm', 'd'):
        a, b = _DST_OF[t]
        wr = 0.5 * (P1[a]['Wr'] + P1[b]['Wr'])
        bb = (0.5 * (P1[a]['bl'] + P1[b]['bl'])).reshape(1, _DH)
        ra, rb = [r for r in _REL if _SRCTY[r] == t]
        H[t], y2[ra], y2[rb] = _combine1(
            S1[a], S1[b], X[t], wr, bb, P2[ra]['Wl'], P2[rb]['Wl'])

    args2 = ([y2[r] for r in _REL]
             + [idx[r][0] for r in _REL]
             + [idx[r][1] for r in _REL])
    outs2 = _seg_l2(*args2)
    S2 = {r: outs2[i] for i, r in enumerate(_REL)}

    O = {}
    for t in ('c', 'm', 'd'):
        a, b = _DST_OF[t]
        wr = 0.5 * (P2[a]['Wr'] + P2[b]['Wr'])
        bb = (0.5 * (P2[a]['bl'] + P2[b]['bl']) + Pr[t]['b']).reshape(1, _DH)
        O[t] = _combine2(S2[a], S1[a], S2[b], S1[b], H[t], X[t],
                         wr, Pr[t]['W'], bb)

    return O['c'], O['m'], O['d']
